# baseline (speedup 1.0000x reference)
import functools

import jax
import jax.numpy as jnp
from jax import lax
from jax.experimental import pallas as pl
from jax.experimental.pallas import tpu as pltpu
from jax.experimental.pallas import tpu_sc as plsc

N_CORE = 20000
N_AUX = 5000
N_BINS = N_CORE + N_AUX
BINS_PAD = 26624
D = 256
N_IDS = 1024000
NUM_WORKERS = 32
IDS_PER_W = N_IDS // NUM_WORKERS
LANES = 16

_mesh = plsc.VectorSubcoreMesh(core_axis_name="c", subcore_axis_name="s")

NUM_SUB = 16
SL = BINS_PAD // NUM_SUB


@functools.partial(
    pl.kernel,
    out_type=jax.ShapeDtypeStruct((2, BINS_PAD), jnp.float32),
    mesh=_mesh,
    scratch_types=[
        pltpu.VMEM((IDS_PER_W,), jnp.int32),
        pltpu.VMEM((BINS_PAD,), jnp.float32),
        pltpu.VMEM((NUM_SUB, SL), jnp.float32),
        pltpu.VMEM_SHARED((NUM_SUB, BINS_PAD), jnp.float32),
        pltpu.SemaphoreType.DMA,
    ],
    compiler_params=pltpu.CompilerParams(needs_layout_passes=False),
)
def _histogram(ids_hbm, out_hbm, ids_v, counts_v, red_v, shared, sem):
    cid = lax.axis_index("c")
    sid = lax.axis_index("s")
    wid = sid * 2 + cid
    cp = pltpu.async_copy(
        ids_hbm.at[pl.ds(wid * IDS_PER_W, IDS_PER_W)], ids_v, sem
    )

    zeros = jnp.zeros((LANES,), jnp.float32)

    @plsc.parallel_loop(0, BINS_PAD // LANES, unroll=8)
    def _zero(i):
        counts_v[pl.ds(i * LANES, LANES)] = zeros

    cp.wait()
    ones = jnp.ones((LANES,), jnp.float32)

    @plsc.parallel_loop(0, IDS_PER_W // LANES, unroll=8)
    def _scat(i):
        idx = ids_v[pl.ds(i * LANES, LANES)]
        plsc.addupdate_scatter(counts_v, [idx], ones)

    pltpu.sync_copy(counts_v, shared.at[sid])
    plsc.subcore_barrier()
    base = sid * SL
    pltpu.sync_copy(shared.at[:, pl.ds(base, SL)], red_v)

    @plsc.parallel_loop(0, SL // LANES, unroll=2)
    def _red(i):
        acc = red_v[0, pl.ds(i * LANES, LANES)]
        for r in range(1, NUM_SUB):
            acc = acc + red_v[r, pl.ds(i * LANES, LANES)]
        counts_v[pl.ds(i * LANES, LANES)] = acc

    pltpu.sync_copy(counts_v.at[pl.ds(0, SL)],
                    out_hbm.at[cid, pl.ds(base, SL)])


_CB = 4000
_AB = 1000
_STEPS = 5


def _matvec_body(cc_ref, ca_ref, core_ref, aux_ref, out_ref):
    j = pl.program_id(0)

    @pl.when(j == 0)
    def _init():
        out_ref[...] = jnp.zeros_like(out_ref)

    cc = jnp.sum(cc_ref[...], axis=1, keepdims=True)
    ca = jnp.sum(ca_ref[...], axis=1, keepdims=True)
    acc = lax.dot_general(
        cc, core_ref[...], (((0,), (0,)), ((), ())),
        precision=lax.Precision.HIGHEST,
        preferred_element_type=jnp.float32,
    )
    acc = acc + lax.dot_general(
        ca, aux_ref[...], (((0,), (0,)), ((), ())),
        precision=lax.Precision.HIGHEST,
        preferred_element_type=jnp.float32,
    )
    out_ref[...] += acc

    @pl.when(j == _STEPS - 1)
    def _fin():
        out_ref[...] *= (1.0 / N_IDS)


def _weighted_sum(counts_t, core_feats, aux_feats):
    return pl.pallas_call(
        _matvec_body,
        grid=(_STEPS,),
        in_specs=[
            pl.BlockSpec((_CB, 2), lambda j: (j, 0)),
            pl.BlockSpec((_AB, 2), lambda j: (N_CORE // _AB + j, 0)),
            pl.BlockSpec((_CB, D), lambda j: (j, 0)),
            pl.BlockSpec((_AB, D), lambda j: (j, 0)),
        ],
        out_specs=pl.BlockSpec((1, D), lambda j: (0, 0)),
        out_shape=jax.ShapeDtypeStruct((1, D), jnp.float32),
    )(counts_t, counts_t, core_feats, aux_feats)


def kernel(core_feats, aux_feats, id_map):
    ids = id_map.reshape(-1).astype(jnp.int32)
    counts = _histogram(ids)
    counts_t = counts.T
    return _weighted_sum(counts_t, core_feats, aux_feats)


# speedup vs baseline: 1.1853x; 1.1853x over previous
import functools

import jax
import jax.numpy as jnp
from jax import lax
from jax.experimental import pallas as pl
from jax.experimental.pallas import tpu as pltpu
from jax.experimental.pallas import tpu_sc as plsc

N_CORE = 20000
N_AUX = 5000
AUX_OFF = 20096
GAP = AUX_OFF - N_CORE
BINS_PAD = 26624
D = 256
N_IDS = 1024000
NUM_WORKERS = 32
LANES = 16
ID_ROWS = N_IDS // 128
W_ROWS = ID_ROWS // NUM_WORKERS

_mesh = plsc.VectorSubcoreMesh(core_axis_name="c", subcore_axis_name="s")

NUM_SUB = 16
SL = BINS_PAD // NUM_SUB


@functools.partial(
    pl.kernel,
    out_type=jax.ShapeDtypeStruct((2, BINS_PAD), jnp.float32),
    mesh=_mesh,
    scratch_types=[
        pltpu.VMEM((N_IDS // NUM_WORKERS,), jnp.int32),
        pltpu.VMEM((BINS_PAD,), jnp.float32),
        pltpu.VMEM((NUM_SUB, SL), jnp.float32),
        pltpu.VMEM_SHARED((NUM_SUB, BINS_PAD), jnp.float32),
        pltpu.SemaphoreType.DMA,
    ],
    compiler_params=pltpu.CompilerParams(needs_layout_passes=False),
)
def _histogram(ids_hbm, out_hbm, ids_v, counts_v, red_v, shared, sem):
    cid = lax.axis_index("c")
    sid = lax.axis_index("s")
    wid = sid * 2 + cid
    cp = pltpu.async_copy(
        ids_hbm.at[pl.ds(wid * (N_IDS // NUM_WORKERS), N_IDS // NUM_WORKERS)],
        ids_v, sem,
    )

    zeros = jnp.zeros((LANES,), jnp.float32)

    @plsc.parallel_loop(0, BINS_PAD // LANES, unroll=8)
    def _zero(i):
        counts_v[pl.ds(i * LANES, LANES)] = zeros

    cp.wait()
    ones = jnp.ones((LANES,), jnp.float32)

    @plsc.parallel_loop(0, N_IDS // NUM_WORKERS // LANES, unroll=8)
    def _scat(i):
        idx = ids_v[pl.ds(i * LANES, LANES)]
        idx = jnp.where(idx >= N_CORE, idx + GAP, idx)
        plsc.addupdate_scatter(counts_v, [idx], ones)

    pltpu.sync_copy(counts_v, shared.at[sid])
    plsc.subcore_barrier()
    base = sid * SL
    pltpu.sync_copy(shared.at[:, pl.ds(base, SL)], red_v)

    @plsc.parallel_loop(0, SL // LANES, unroll=2)
    def _red(i):
        acc = red_v[0, pl.ds(i * LANES, LANES)]
        for r in range(1, NUM_SUB):
            acc = acc + red_v[r, pl.ds(i * LANES, LANES)]
        counts_v[pl.ds(i * LANES, LANES)] = acc

    pltpu.sync_copy(counts_v.at[pl.ds(0, SL)],
                    out_hbm.at[cid, pl.ds(base, SL)])


_CB = 4096
_AB = 1024
_STEPS = 5


def _matvec_body(cnt_ref, core_ref, aux_ref, out_ref, acc_ref):
    j = pl.program_id(0)

    @pl.when(j == 0)
    def _init():
        acc_ref[...] = jnp.zeros_like(acc_ref)

    cc = cnt_ref[:, pl.ds(j * _CB, _CB)]
    ca = cnt_ref[:, pl.ds(AUX_OFF + j * _AB, _AB)]

    core_rows = j * _CB + lax.broadcasted_iota(jnp.int32, (_CB, 1), 0)
    core = jnp.where(core_rows < N_CORE, core_ref[...], 0.0)
    aux_rows = j * _AB + lax.broadcasted_iota(jnp.int32, (_AB, 1), 0)
    aux = jnp.where(aux_rows < N_AUX, aux_ref[...], 0.0)

    acc = lax.dot_general(
        cc, core, (((1,), (0,)), ((), ())),
        precision=lax.Precision.HIGHEST,
        preferred_element_type=jnp.float32,
    )
    acc = acc + lax.dot_general(
        ca, aux, (((1,), (0,)), ((), ())),
        precision=lax.Precision.HIGHEST,
        preferred_element_type=jnp.float32,
    )
    acc_ref[...] += acc

    @pl.when(j == _STEPS - 1)
    def _fin():
        out_ref[...] = (acc_ref[0:1, :] + acc_ref[1:2, :]) * (1.0 / N_IDS)


def _weighted_sum(counts, core_feats, aux_feats):
    return pl.pallas_call(
        _matvec_body,
        grid=(_STEPS,),
        in_specs=[
            pl.BlockSpec((2, BINS_PAD), lambda j: (0, 0)),
            pl.BlockSpec((_CB, D), lambda j: (j, 0)),
            pl.BlockSpec((_AB, D), lambda j: (j, 0)),
        ],
        out_specs=pl.BlockSpec((1, D), lambda j: (0, 0)),
        out_shape=jax.ShapeDtypeStruct((1, D), jnp.float32),
        scratch_shapes=[pltpu.VMEM((2, D), jnp.float32)],
    )(counts, core_feats, aux_feats)


def kernel(core_feats, aux_feats, id_map):
    ids = id_map.reshape(-1)
    counts = _histogram(ids)
    return _weighted_sum(counts, core_feats, aux_feats)
